# baseline (speedup 1.0000x reference)
import functools

import jax
import jax.numpy as jnp
from jax.experimental import pallas as pl


def _prep_kernel(ifb_ref, iast_ref, exp_u_ref, tle_ref, bnd_ref, r_ref,
                 t_ref, f_ref, big_ref, base_ref):
    r = r_ref[0, 0]
    tle = tle_ref[0, 0]
    bnd = bnd_ref[0, 0]
    bounds = jnp.max(jnp.sum(ifb_ref[...], axis=-1)) * 5.0
    sr = bounds * r

    uc = jnp.clip(exp_u_ref[...], 0.0, 1.0 - 1e-7)
    e = -jnp.log1p(-uc) / sr
    e2 = e.reshape(64, 128)
    i0 = jax.lax.broadcasted_iota(jnp.int32, (128, 128), 0)
    i1 = jax.lax.broadcasted_iota(jnp.int32, (128, 128), 1)
    upper = (i0 <= i1).astype(jnp.float32)
    cums = jax.lax.dot(e2, upper, preferred_element_type=jnp.float32)
    totals = cums[:, 127:128]
    j0 = jax.lax.broadcasted_iota(jnp.int32, (64, 64), 0)
    j1 = jax.lax.broadcasted_iota(jnp.int32, (64, 64), 1)
    strict = (j1 < j0).astype(jnp.float32)
    offs = jax.lax.dot(strict, totals, preferred_element_type=jnp.float32)
    t2 = cums + offs + tle
    t = t2.reshape(1, -1)
    t_ref[...] = t

    ti = jnp.sum(iast_ref[...], axis=-1) * r
    f_ref[...] = sr / ti

    big = jnp.max(t2) + 1.0
    t_last = t[0, t.shape[1] - 1]
    big_ref[...] = jnp.full((1, 1), big)
    base_ref[...] = jnp.full((1, 1), jnp.where(t_last > bnd, t_last, bnd))


def _scan_kernel(u_ref, t_ref, f_ref, big_ref, base_ref, rst_ref, w_ref):
    u = u_ref[...]
    c = u * f_ref[...]
    mask = c < 1.0
    tacc = jnp.where(mask, t_ref[...], big_ref[...])
    m = jnp.min(tacc, axis=1, keepdims=True)
    minc = jnp.min(c, axis=1, keepdims=True)
    rst_ref[...] = jnp.where(minc < 1.0, m, base_ref[...])
    w_ref[...] = jnp.full(w_ref.shape, 1.0 / (w_ref.shape[0] * pl.num_programs(0)),
                          dtype=jnp.float32)


def kernel(intensities_for_bound, intensities_at_sampled_times, exp_u,
           unif_numbers, time_last_event, boundary, ratio):
    num_sample, S = unif_numbers.shape
    tle = time_last_event.reshape(1, 1)
    bnd = boundary.reshape(1, 1)
    r = ratio.reshape(1, 1)

    t, f, big, base = pl.pallas_call(
        _prep_kernel,
        out_shape=(
            jax.ShapeDtypeStruct((1, S), jnp.float32),
            jax.ShapeDtypeStruct((1, S), jnp.float32),
            jax.ShapeDtypeStruct((1, 1), jnp.float32),
            jax.ShapeDtypeStruct((1, 1), jnp.float32),
        ),
    )(intensities_for_bound, intensities_at_sampled_times, exp_u, tle, bnd, r)

    B = 128
    grid = (num_sample // B,)
    rst, w = pl.pallas_call(
        _scan_kernel,
        grid=grid,
        in_specs=[
            pl.BlockSpec((B, S), lambda i: (i, 0)),
            pl.BlockSpec((1, S), lambda i: (0, 0)),
            pl.BlockSpec((1, S), lambda i: (0, 0)),
            pl.BlockSpec((1, 1), lambda i: (0, 0)),
            pl.BlockSpec((1, 1), lambda i: (0, 0)),
        ],
        out_specs=(
            pl.BlockSpec((B, 1), lambda i: (i, 0)),
            pl.BlockSpec((B, 1), lambda i: (i, 0)),
        ),
        out_shape=(
            jax.ShapeDtypeStruct((num_sample, 1), jnp.float32),
            jax.ShapeDtypeStruct((num_sample, 1), jnp.float32),
        ),
    )(unif_numbers, t, f, big, base)
    return (rst.reshape(num_sample), w.reshape(num_sample))


# speedup vs baseline: 1.6478x; 1.6478x over previous
import functools

import jax
import jax.numpy as jnp
from jax import lax
from jax.experimental import pallas as pl
from jax.experimental.pallas import tpu as pltpu
from jax.experimental.pallas import tpu_sc as plsc

_S = 8192
_N = 4096
_C0 = 512
_CHUNK = 512
_NW = 32
_ROWS = _N // _NW


def _prep_kernel(ifb_ref, iast_ref, exp_u_ref, tle_ref, bnd_ref, r_ref,
                 t_ref, th_ref, big_ref, base_ref):
    r = r_ref[0, 0]
    tle = tle_ref[0, 0]
    bnd = bnd_ref[0, 0]
    bounds = jnp.max(jnp.sum(ifb_ref[...], axis=-1)) * 5.0
    sr = bounds * r

    uc = jnp.clip(exp_u_ref[...], 0.0, 1.0 - 1e-7)
    e = -jnp.log1p(-uc) / sr
    e2 = e.reshape(64, 128)
    i0 = lax.broadcasted_iota(jnp.int32, (128, 128), 0)
    i1 = lax.broadcasted_iota(jnp.int32, (128, 128), 1)
    upper = (i0 <= i1).astype(jnp.float32)
    cums = lax.dot(e2, upper, preferred_element_type=jnp.float32)
    totals = cums[:, 127:128]
    j0 = lax.broadcasted_iota(jnp.int32, (64, 64), 0)
    j1 = lax.broadcasted_iota(jnp.int32, (64, 64), 1)
    strict = (j1 < j0).astype(jnp.float32)
    offs = lax.dot(strict, totals, preferred_element_type=jnp.float32)
    t2 = cums + offs + tle
    t = t2.reshape(1, -1)
    t_ref[...] = t

    ti = jnp.sum(iast_ref[...], axis=-1) * r
    th_ref[...] = ti / sr

    big = jnp.max(t2) + 1.0
    t_last = t[0, t.shape[1] - 1]
    big_ref[...] = jnp.full((1, 16), big)
    base_ref[...] = jnp.full((1, 16), jnp.where(t_last > bnd, t_last, bnd))


def _scalarize(x):
    return x if x.ndim == 0 else x[0]


def _sc_scan(th_hbm, t_hbm, big_hbm, base_hbm, u_hbm, rst_hbm, w_hbm,
             th_v, t_v, big_v, base_v, ub_v, urow_v, rst_v, sem):
    wid = lax.axis_index("s") * 2 + lax.axis_index("c")
    base_row = wid * _ROWS

    pltpu.async_copy(th_hbm, th_v, sem).wait()
    pltpu.async_copy(t_hbm, t_v, sem).wait()
    pltpu.async_copy(big_hbm, big_v, sem).wait()
    pltpu.async_copy(base_hbm, base_v, sem).wait()
    pltpu.async_copy(
        u_hbm.at[pl.ds(base_row, _ROWS), pl.ds(0, _C0)], ub_v, sem).wait()

    big_s = _scalarize(big_v[...])
    base_s = _scalarize(base_v[...])
    lanes = lax.broadcasted_iota(jnp.int32, (16,), 0)

    def scan_chunks(j0, j1, load_u):
        def cond(c):
            j, found, _ = c
            return jnp.logical_and(jnp.logical_not(found), j < j1)

        def body(c):
            j, _, val = c
            u16 = load_u(j)
            th16 = th_v[pl.ds(j * 16, 16)]
            mask = u16 < th16
            pc = plsc.all_reduce_population_count(mask)
            anyacc = _scalarize(pc) > 0
            idx = plsc.all_reduce_ffs(mask)
            idxv = idx if idx.ndim == 1 else jnp.full((16,), idx)
            idxv = jnp.minimum(idxv, 15)
            tsel = plsc.load_gather(t_v, [j * 16 + idxv])
            val = jnp.where(anyacc, _scalarize(tsel), val)
            return (j + 1, anyacc, val)

        _, found, val = lax.while_loop(
            cond, body, (j0, jnp.bool_(False), big_s))
        return found, val

    def row_body(r, accv):
        def load_staged(j):
            return plsc.load_gather(
                ub_v, [jnp.full((16,), r, jnp.int32), j * 16 + lanes])

        found, val = scan_chunks(jnp.int32(0), jnp.int32(_C0 // 16),
                                 load_staged)

        def fb_cond(c):
            k, found, _ = c
            return jnp.logical_and(jnp.logical_not(found), k < _S // _CHUNK)

        def fb_body(c):
            k, _, _ = c
            pltpu.async_copy(
                u_hbm.at[base_row + r, pl.ds(k * _CHUNK, _CHUNK)],
                urow_v, sem).wait()

            def load_fb(j):
                return urow_v[pl.ds((j - k * (_CHUNK // 16)) * 16, 16)]

            found, val = scan_chunks(k * (_CHUNK // 16),
                                     (k + 1) * (_CHUNK // 16), load_fb)
            return (k + 1, found, val)

        _, found, val = lax.while_loop(
            fb_cond, fb_body, (jnp.int32(_C0 // _CHUNK), found, val))

        out = jnp.where(found, val, base_s)
        accv = jnp.where(lanes == r % 16, jnp.full((16,), out), accv)

        @pl.when(r % 16 == 15)
        def _():
            rst_v[pl.ds((r // 16) * 16, 16)] = accv

        return accv

    lax.fori_loop(0, _ROWS, row_body, jnp.zeros((16,), jnp.float32))

    pltpu.async_copy(rst_v, rst_hbm.at[pl.ds(base_row, _ROWS)], sem).wait()
    w = jnp.full((16,), 1.0 / _N, jnp.float32)
    for g in range(_ROWS // 16):
        rst_v[pl.ds(g * 16, 16)] = w
    pltpu.async_copy(rst_v, w_hbm.at[pl.ds(base_row, _ROWS)], sem).wait()


def kernel(intensities_for_bound, intensities_at_sampled_times, exp_u,
           unif_numbers, time_last_event, boundary, ratio):
    num_sample, S = unif_numbers.shape
    tle = time_last_event.reshape(1, 1)
    bnd = boundary.reshape(1, 1)
    r = ratio.reshape(1, 1)

    t, th, bigv, basev = pl.pallas_call(
        _prep_kernel,
        out_shape=(
            jax.ShapeDtypeStruct((1, S), jnp.float32),
            jax.ShapeDtypeStruct((1, S), jnp.float32),
            jax.ShapeDtypeStruct((1, 16), jnp.float32),
            jax.ShapeDtypeStruct((1, 16), jnp.float32),
        ),
    )(intensities_for_bound, intensities_at_sampled_times, exp_u, tle, bnd, r)

    mesh = plsc.VectorSubcoreMesh(core_axis_name="c", subcore_axis_name="s")
    sck = functools.partial(
        pl.kernel,
        mesh=mesh,
        compiler_params=pltpu.CompilerParams(needs_layout_passes=False),
        out_type=(
            jax.ShapeDtypeStruct((num_sample,), jnp.float32),
            jax.ShapeDtypeStruct((num_sample,), jnp.float32),
        ),
        scratch_types=[
            pltpu.VMEM((S,), jnp.float32),
            pltpu.VMEM((S,), jnp.float32),
            pltpu.VMEM((16,), jnp.float32),
            pltpu.VMEM((16,), jnp.float32),
            pltpu.VMEM((_ROWS, _C0), jnp.float32),
            pltpu.VMEM((_CHUNK,), jnp.float32),
            pltpu.VMEM((_ROWS,), jnp.float32),
            pltpu.SemaphoreType.DMA,
        ],
    )(_sc_scan)
    rst, w = sck(th.reshape(S), t.reshape(S), bigv.reshape(16),
                 basev.reshape(16), unif_numbers)
    return (rst, w)


# speedup vs baseline: 1.8305x; 1.1109x over previous
import functools

import jax
import jax.numpy as jnp
from jax import lax
from jax.experimental import pallas as pl
from jax.experimental.pallas import tpu as pltpu
from jax.experimental.pallas import tpu_sc as plsc

_S = 8192
_N = 4096
_C0 = 128
_CHUNK = 128
_NW = 32
_ROWS = _N // _NW


def _prep_kernel(ifb_ref, iast_ref, exp_u_ref, tle_ref, bnd_ref, r_ref,
                 t_ref, th_ref, base_ref):
    r = r_ref[0, 0]
    tle = tle_ref[0, 0]
    bnd = bnd_ref[0, 0]
    bounds = jnp.max(jnp.sum(ifb_ref[...], axis=-1)) * 5.0
    sr = bounds * r

    uc = jnp.clip(exp_u_ref[...], 0.0, 1.0 - 1e-7)
    e = -jnp.log1p(-uc) / sr
    e2 = e.reshape(64, 128)
    i0 = lax.broadcasted_iota(jnp.int32, (128, 128), 0)
    i1 = lax.broadcasted_iota(jnp.int32, (128, 128), 1)
    upper = (i0 <= i1).astype(jnp.float32)
    cums = lax.dot(e2, upper, preferred_element_type=jnp.float32)
    totals = cums[:, 127:128]
    j0 = lax.broadcasted_iota(jnp.int32, (64, 64), 0)
    j1 = lax.broadcasted_iota(jnp.int32, (64, 64), 1)
    strict = (j1 < j0).astype(jnp.float32)
    offs = lax.dot(strict, totals, preferred_element_type=jnp.float32)
    t2 = cums + offs + tle
    t = t2.reshape(1, -1)
    t_ref[...] = t

    ti = jnp.sum(iast_ref[...], axis=-1) * r
    th_ref[...] = ti / sr

    t_last = t[0, t.shape[1] - 1]
    base_ref[...] = jnp.full((1, 16), jnp.where(t_last > bnd, t_last, bnd))


def _scalarize(x):
    return x if x.ndim == 0 else x[0]


def _sc_scan(th_hbm, t_hbm, base_hbm, u_hbm, rst_hbm, w_hbm,
             th_v, t_v, base_v, ub_v, urow_v, rst_v, sem):
    wid = lax.axis_index("s") * 2 + lax.axis_index("c")
    base_row = wid * _ROWS

    c1 = pltpu.async_copy(th_hbm, th_v, sem)
    c2 = pltpu.async_copy(t_hbm, t_v.at[pl.ds(0, _S)], sem)
    c3 = pltpu.async_copy(base_hbm, base_v, sem)
    c4 = pltpu.async_copy(
        u_hbm.at[pl.ds(base_row, _ROWS), pl.ds(0, _C0)], ub_v, sem)
    c1.wait()
    c2.wait()
    c3.wait()
    c4.wait()

    base_s = _scalarize(base_v[...])
    lanes = lax.broadcasted_iota(jnp.int32, (16,), 0)

    def scan_chunks(j0, j1, sel0, load_u):
        def cond(c):
            j, found, _ = c
            return jnp.logical_and(jnp.logical_not(found), j < j1)

        def body(c):
            j, _, sel = c
            u16 = load_u(j)
            th16 = th_v[pl.ds(j * 16, 16)]
            ffs = _scalarize(plsc.all_reduce_ffs(u16 < th16))
            found = ffs < 16
            sel = jnp.where(found, j * 16 + ffs, sel)
            return (j + 1, found, sel)

        _, found, sel = lax.while_loop(
            cond, body, (j0, jnp.bool_(False), sel0))
        return found, sel

    def row_body(r, accv):
        def load_staged(j):
            return plsc.load_gather(
                ub_v, [jnp.full((16,), r, jnp.int32), j * 16 + lanes])

        found, sel = scan_chunks(jnp.int32(0), jnp.int32(_C0 // 16),
                                 jnp.int32(0), load_staged)

        def fb_cond(c):
            k, found, _ = c
            return jnp.logical_and(jnp.logical_not(found), k < _S // _CHUNK)

        def fb_body(c):
            k, _, sel_in = c
            pltpu.async_copy(
                u_hbm.at[base_row + r, pl.ds(k * _CHUNK, _CHUNK)],
                urow_v, sem).wait()

            def load_fb(j):
                return urow_v[pl.ds((j - k * (_CHUNK // 16)) * 16, 16)]

            found, sel = scan_chunks(k * (_CHUNK // 16),
                                     (k + 1) * (_CHUNK // 16), sel_in, load_fb)
            return (k + 1, found, sel)

        _, found, sel = lax.while_loop(
            fb_cond, fb_body, (jnp.int32(_C0 // _CHUNK), found, sel))

        val = _scalarize(t_v[pl.ds(sel, 16)])
        out = jnp.where(found, val, base_s)
        accv = jnp.where(lanes == (r & 15), jnp.full((16,), out), accv)

        @pl.when((r & 15) == 15)
        def _():
            rst_v[pl.ds((r >> 4) * 16, 16)] = accv

        return accv

    lax.fori_loop(0, _ROWS, row_body, jnp.zeros((16,), jnp.float32))

    pltpu.async_copy(rst_v, rst_hbm.at[pl.ds(base_row, _ROWS)], sem).wait()
    w = jnp.full((16,), 1.0 / _N, jnp.float32)
    for g in range(_ROWS // 16):
        rst_v[pl.ds(g * 16, 16)] = w
    pltpu.async_copy(rst_v, w_hbm.at[pl.ds(base_row, _ROWS)], sem).wait()


def kernel(intensities_for_bound, intensities_at_sampled_times, exp_u,
           unif_numbers, time_last_event, boundary, ratio):
    num_sample, S = unif_numbers.shape
    tle = time_last_event.reshape(1, 1)
    bnd = boundary.reshape(1, 1)
    r = ratio.reshape(1, 1)

    t, th, basev = pl.pallas_call(
        _prep_kernel,
        out_shape=(
            jax.ShapeDtypeStruct((1, S), jnp.float32),
            jax.ShapeDtypeStruct((1, S), jnp.float32),
            jax.ShapeDtypeStruct((1, 16), jnp.float32),
        ),
    )(intensities_for_bound, intensities_at_sampled_times, exp_u, tle, bnd, r)

    mesh = plsc.VectorSubcoreMesh(core_axis_name="c", subcore_axis_name="s")
    sck = functools.partial(
        pl.kernel,
        mesh=mesh,
        compiler_params=pltpu.CompilerParams(needs_layout_passes=False),
        out_type=(
            jax.ShapeDtypeStruct((num_sample,), jnp.float32),
            jax.ShapeDtypeStruct((num_sample,), jnp.float32),
        ),
        scratch_types=[
            pltpu.VMEM((S,), jnp.float32),
            pltpu.VMEM((S + 16,), jnp.float32),
            pltpu.VMEM((16,), jnp.float32),
            pltpu.VMEM((_ROWS, _C0), jnp.float32),
            pltpu.VMEM((_CHUNK,), jnp.float32),
            pltpu.VMEM((_ROWS,), jnp.float32),
            pltpu.SemaphoreType.DMA,
        ],
    )(_sc_scan)
    rst, w = sck(th.reshape(S), t.reshape(S), basev.reshape(16), unif_numbers)
    return (rst, w)


# speedup vs baseline: 2.0931x; 1.1435x over previous
import functools

import jax
import jax.numpy as jnp
from jax import lax
from jax.experimental import pallas as pl
from jax.experimental.pallas import tpu as pltpu
from jax.experimental.pallas import tpu_sc as plsc

_S = 8192
_N = 4096
_C0 = 128
_CHUNK = 128
_NW = 32
_ROWS = _N // _NW
_UBSTRIDE = _C0 + 1


def _prep_kernel(ifb_ref, iast_ref, exp_u_ref, tle_ref, bnd_ref, r_ref,
                 t_ref, th_ref):
    r = r_ref[0, 0]
    tle = tle_ref[0, 0]
    bnd = bnd_ref[0, 0]
    bounds = jnp.max(jnp.sum(ifb_ref[...], axis=-1)) * 5.0
    sr = bounds * r

    uc = jnp.clip(exp_u_ref[...], 0.0, 1.0 - 1e-7)
    e2 = -jnp.log1p(-uc) / sr
    i0 = lax.broadcasted_iota(jnp.int32, (128, 128), 0)
    i1 = lax.broadcasted_iota(jnp.int32, (128, 128), 1)
    upper = (i0 <= i1).astype(jnp.float32)
    cums = lax.dot(e2, upper, precision=lax.Precision.HIGHEST,
                   preferred_element_type=jnp.float32)
    totals = cums[:, 127:128]
    j0 = lax.broadcasted_iota(jnp.int32, (64, 64), 0)
    j1 = lax.broadcasted_iota(jnp.int32, (64, 64), 1)
    strict = (j1 < j0).astype(jnp.float32)
    offs = lax.dot(strict, totals, precision=lax.Precision.HIGHEST,
                   preferred_element_type=jnp.float32)
    t2 = cums + offs + tle

    t_last = t2[63, 127]
    base = jnp.where(t_last > bnd, t_last, bnd)
    t_ref[0:64, :] = t2
    t_ref[64:72, :] = jnp.full((8, 128), base)

    k0 = lax.broadcasted_iota(jnp.int32, (1024, 128), 0)
    k1 = lax.broadcasted_iota(jnp.int32, (1024, 128), 1)
    sel = ((k0 >> 3) == k1).astype(jnp.float32)
    ti = lax.dot(iast_ref[...], sel, precision=lax.Precision.HIGHEST,
                 preferred_element_type=jnp.float32)
    th_ref[...] = ti * (r / sr)


def _scalarize(x):
    return x if x.ndim == 0 else x[0]


def _sc_scan(th_hbm, t_hbm, u_hbm, rst_hbm, w_hbm,
             th_v, t_v, ub_v, urow_v, rst_v, done_v, sem):
    wid = lax.axis_index("s") * 2 + lax.axis_index("c")
    base_row = wid * _ROWS

    c1 = pltpu.async_copy(th_hbm, th_v, sem)
    c2 = pltpu.async_copy(t_hbm, t_v, sem)
    c3 = pltpu.async_copy(
        u_hbm.at[pl.ds(base_row, _ROWS), pl.ds(0, _C0)], ub_v, sem)
    c1.wait()
    c2.wait()
    c3.wait()

    lanes = lax.broadcasted_iota(jnp.int32, (16,), 0)
    base_splat = jnp.full((16,), _scalarize(t_v[pl.ds(_S, 16)]))

    def group_body(g, und):
        rowids = g * 16 + lanes

        def cond(c):
            cc, alldone = c[0], c[1]
            return jnp.logical_and(jnp.logical_not(alldone), cc < _C0)

        def body(c):
            cc, _, done, colsel = c
            thch = th_v[pl.ds(cc, 16)]
            for s in range(8):
                col = cc + s
                u_c = plsc.load_gather(
                    ub_v, [rowids, jnp.full((16,), col, jnp.int32)])
                th_c = jnp.full((16,), thch[s])
                mask = u_c < th_c
                newly = jnp.logical_and(mask, jnp.logical_not(done))
                colsel = jnp.where(
                    newly, jnp.full((16,), col, jnp.int32), colsel)
                done = jnp.logical_or(done, mask)
            nd = _scalarize(plsc.all_reduce_population_count(done))
            return (cc + 8, nd == 16, done, colsel)

        _, _, done, colsel = lax.while_loop(
            cond, body,
            (jnp.int32(0), jnp.bool_(False),
             jnp.zeros((16,), jnp.bool_), jnp.zeros((16,), jnp.int32)))

        times = plsc.load_gather(t_v, [colsel])
        rst_v[pl.ds(g * 16, 16)] = jnp.where(done, times, base_splat)
        done_v[pl.ds(g * 16, 16)] = done.astype(jnp.int32)
        nd = _scalarize(plsc.all_reduce_population_count(done))
        return und + (16 - nd)

    und = lax.fori_loop(0, _ROWS // 16, group_body, jnp.int32(0))

    def scan_chunks(j0, j1, sel0, load_u):
        def cond(c):
            j, found = c[0], c[1]
            return jnp.logical_and(jnp.logical_not(found), j < j1)

        def body(c):
            j, _, sel = c
            u16 = load_u(j)
            th16 = th_v[pl.ds(j * 16, 16)]
            ffs = _scalarize(plsc.all_reduce_ffs(u16 < th16))
            found = ffs < 16
            sel = jnp.where(found, j * 16 + ffs, sel)
            return (j + 1, found, sel)

        _, found, sel = lax.while_loop(
            cond, body, (j0, jnp.bool_(False), sel0))
        return found, sel

    @pl.when(und > 0)
    def _():
        def row_body(r, carry):
            fnd = _scalarize(plsc.load_gather(
                done_v, [jnp.full((16,), r, jnp.int32)]))

            @pl.when(fnd == 0)
            def _():
                def fb_cond(c):
                    k, found = c[0], c[1]
                    return jnp.logical_and(
                        jnp.logical_not(found), k < _S // _CHUNK)

                def fb_body(c):
                    k, _, sel_in = c
                    pltpu.async_copy(
                        u_hbm.at[base_row + r, pl.ds(k * _CHUNK, _CHUNK)],
                        urow_v, sem).wait()

                    def load_fb(j):
                        return urow_v[pl.ds((j - k * (_CHUNK // 16)) * 16, 16)]

                    found, sel = scan_chunks(
                        k * (_CHUNK // 16), (k + 1) * (_CHUNK // 16),
                        sel_in, load_fb)
                    return (k + 1, found, sel)

                _, found, sel = lax.while_loop(
                    fb_cond, fb_body,
                    (jnp.int32(_C0 // _CHUNK), jnp.bool_(False), jnp.int32(0)))

                @pl.when(found)
                def _():
                    val = _scalarize(t_v[pl.ds(sel, 16)])
                    plsc.store_scatter(
                        rst_v, [jnp.full((16,), r, jnp.int32)],
                        jnp.full((16,), val), mask=lanes == 0)

            return carry

        lax.fori_loop(0, _ROWS, row_body, jnp.int32(0))

    pltpu.async_copy(rst_v, rst_hbm.at[pl.ds(base_row, _ROWS)], sem).wait()
    w = jnp.full((16,), 1.0 / _N, jnp.float32)
    for g in range(_ROWS // 16):
        rst_v[pl.ds(g * 16, 16)] = w
    pltpu.async_copy(rst_v, w_hbm.at[pl.ds(base_row, _ROWS)], sem).wait()


def kernel(intensities_for_bound, intensities_at_sampled_times, exp_u,
           unif_numbers, time_last_event, boundary, ratio):
    num_sample, S = unif_numbers.shape
    tle = time_last_event.reshape(1, 1)
    bnd = boundary.reshape(1, 1)
    r = ratio.reshape(1, 1)

    t72, th64 = pl.pallas_call(
        _prep_kernel,
        out_shape=(
            jax.ShapeDtypeStruct((72, 128), jnp.float32),
            jax.ShapeDtypeStruct((64, 128), jnp.float32),
        ),
    )(intensities_for_bound,
      intensities_at_sampled_times.reshape(64, 1024),
      exp_u.reshape(64, 128), tle, bnd, r)

    mesh = plsc.VectorSubcoreMesh(core_axis_name="c", subcore_axis_name="s")
    sck = functools.partial(
        pl.kernel,
        mesh=mesh,
        compiler_params=pltpu.CompilerParams(needs_layout_passes=False),
        out_type=(
            jax.ShapeDtypeStruct((num_sample,), jnp.float32),
            jax.ShapeDtypeStruct((num_sample,), jnp.float32),
        ),
        scratch_types=[
            pltpu.VMEM((S,), jnp.float32),
            pltpu.VMEM((72 * 128,), jnp.float32),
            pltpu.VMEM((_ROWS, _C0), jnp.float32),
            pltpu.VMEM((_CHUNK,), jnp.float32),
            pltpu.VMEM((_ROWS,), jnp.float32),
            pltpu.VMEM((_ROWS,), jnp.int32),
            pltpu.SemaphoreType.DMA,
        ],
    )(_sc_scan)
    rst, w = sck(th64.reshape(S), t72.reshape(72 * 128), unif_numbers)
    return (rst, w)
